# Initial kernel scaffold; baseline (speedup 1.0000x reference)
import jax
import jax.numpy as jnp
from jax.experimental import pallas as pl


def kernel(images, labels, index):
    raise NotImplementedError("write your pallas kernel here")


# speedup vs baseline: 40.3062x; 40.3062x over previous
import functools

import jax
import jax.numpy as jnp
import numpy as np
from jax.experimental import pallas as pl
from jax.experimental.pallas import tpu as pltpu


def _cut_box(H, W, alpha=1.0, seed=0):
    rng = np.random.RandomState(seed)
    lam = rng.beta(alpha, alpha)
    cx = rng.uniform(0, W)
    cy = rng.uniform(0, H)
    w = W * np.sqrt(1.0 - lam)
    h = H * np.sqrt(1.0 - lam)
    x0 = int(np.clip(cx - w // 2, 0, W))
    y0 = int(np.clip(cy - h // 2, 0, H))
    x1 = int(np.clip(cx + w // 2, 0, W))
    y1 = int(np.clip(cy + h // 2, 0, H))
    return x0, y0, x1, y1


def _mix_body(x0, y0, x1, y1, index_ref, labels_ref, img_ref, perm_ref,
              out_ref, lab_out_ref):
    b = pl.program_id(0)
    out_ref[...] = img_ref[...]
    out_ref[:, :, y0:y1, x0:x1] = perm_ref[:, :, y0:y1, x0:x1]
    lab_out_ref[b] = labels_ref[index_ref[b]]


def kernel(images, labels, index):
    B, C, H, W = images.shape
    x0, y0, x1, y1 = _cut_box(H, W, alpha=1.0, seed=0)

    grid_spec = pltpu.PrefetchScalarGridSpec(
        num_scalar_prefetch=2,
        grid=(B,),
        in_specs=[
            pl.BlockSpec((1, C, H, W), lambda b, idx, lab: (b, 0, 0, 0)),
            pl.BlockSpec((1, C, H, W), lambda b, idx, lab: (idx[b], 0, 0, 0)),
        ],
        out_specs=[
            pl.BlockSpec((1, C, H, W), lambda b, idx, lab: (b, 0, 0, 0)),
            pl.BlockSpec((B,), lambda b, idx, lab: (0,),
                         memory_space=pltpu.SMEM),
        ],
    )
    mixed, labels_b = pl.pallas_call(
        functools.partial(_mix_body, x0, y0, x1, y1),
        grid_spec=grid_spec,
        out_shape=[
            jax.ShapeDtypeStruct(images.shape, images.dtype),
            jax.ShapeDtypeStruct(labels.shape, labels.dtype),
        ],
    )(index, labels, images, images)

    lam = 1.0 - (x1 - x0) * (y1 - y0) / (W * H)
    return (mixed, labels, labels_b, jnp.float32(lam))
